# baseline (speedup 1.0000x reference)
import functools

import jax
import jax.numpy as jnp
from jax import lax
from jax.experimental import pallas as pl
from jax.experimental.pallas import tpu as pltpu
from jax.experimental.pallas import tpu_sc as plsc

B = 128
D_IN = 4096
N_CORES = 128
AXONS = 64
NEURONS = 64
SIM_LEN = 8
MAX_LAT = 3
CYCLES = MAX_LAT + SIM_LEN
N_OUT = 1024
NBUF = N_CORES * NEURONS
AX_TOT = N_CORES * AXONS
BANK_ROWS = D_IN + 8
NWORD = NBUF // 16
CB = 8
GSTEPS = N_CORES // CB

_MESH = plsc.VectorSubcoreMesh(core_axis_name="c", subcore_axis_name="s",
                               num_cores=2, num_subcores=16)


@functools.partial(
    pl.kernel,
    out_type=jax.ShapeDtypeStruct((CYCLES, AX_TOT, B), jnp.float32),
    mesh=_MESH,
    scratch_types=[
        pltpu.VMEM((128,), jnp.int32),
        pltpu.VMEM((128,), jnp.int32),
        pltpu.VMEM((256, B), jnp.float32),
        pltpu.VMEM_SHARED((BANK_ROWS, B), jnp.float32),
    ],
    compiler_params=pltpu.CompilerParams(use_tc_tiling_on_sc=False),
)
def _sc_static(spk_hbm, const_hbm, idx_hbm, out_hbm,
               idx_v0, idx_v1, rows_v, bank):
    tid = lax.axis_index("s")
    wid = tid * 2 + lax.axis_index("c")
    base = wid * (AX_TOT // 32)
    pltpu.sync_copy(idx_hbm.at[pl.ds(base, 128)], idx_v0)
    pltpu.sync_copy(idx_hbm.at[pl.ds(base + 128, 128)], idx_v1)

    def cycle_body(t, carry):
        pltpu.sync_copy(spk_hbm.at[t, pl.ds(tid * 256, 256)],
                        bank.at[pl.ds(tid * 256, 256)])
        @pl.when(tid == 0)
        def _():
            pltpu.sync_copy(const_hbm, bank.at[pl.ds(D_IN, 8)])
        plsc.subcore_barrier()
        pltpu.sync_copy(bank.at[idx_v0], rows_v.at[pl.ds(0, 128)])
        pltpu.sync_copy(bank.at[idx_v1], rows_v.at[pl.ds(128, 128)])
        pltpu.sync_copy(rows_v, out_hbm.at[t, pl.ds(base, 256)])
        plsc.subcore_barrier()
        return carry

    lax.fori_loop(0, CYCLES, cycle_body, 0)


def _tc_body(ins_ref, w_ref, qb_ref, inv2r_ref, qout_ref, oinv_ref,
             pmat_ref, scal_ref, out_ref, bits_ref, memb_ref):
    t = pl.program_id(0)
    g = pl.program_id(1)

    @pl.when(jnp.logical_and(t == 0, g == 0))
    def _():
        out_ref[...] = jnp.zeros((N_OUT, B), jnp.float32)

    pr = (t % 2) * NWORD
    pn = ((t + 1) % 2) * NWORD
    bits_eff = jnp.where(t == 0, 0.0, bits_ref[pl.ds(pr, NWORD), :])
    qb = qb_ref[0]
    words = jax.lax.dot_general(
        qb, bits_eff, (((1,), (0,)), ((), ())),
        precision=lax.Precision.HIGHEST,
        preferred_element_type=jnp.float32)
    v = words * inv2r_ref[0]
    fv = jnp.floor(v)
    buf_bit = fv - 2.0 * jnp.floor(fv * 0.5)
    in3 = ins_ref[0].reshape(CB * AXONS, B) + buf_bit

    pm = pmat_ref[...]
    for ci in range(CB):
        a = scal_ref[0, ci, 0, 0]
        thr = scal_ref[0, ci, 0, 1]
        act = a > 0.5
        x = in3[ci * AXONS:(ci + 1) * AXONS, :]
        delta = jnp.dot(w_ref[0, ci], x, preferred_element_type=jnp.float32)
        mrow = pl.ds(ci * NEURONS + g * CB * NEURONS, NEURONS)
        mold = jnp.where(t == 0, 0.0, memb_ref[mrow, :])
        mn = mold + a * delta
        fb = mn > thr
        fired = jnp.where(fb, 1.0, 0.0)
        memb_ref[mrow, :] = jnp.where(jnp.logical_and(act, fb), 0.0, mn)
        packed = jnp.dot(pm, fired, preferred_element_type=jnp.float32)
        wrow = (g * CB + ci) * 4
        old_w = jnp.where(t == 0, 0.0, bits_ref[pl.ds(pr + wrow, 4), :])
        bits_ref[pl.ds(pn + wrow, 4), :] = jnp.where(act, packed, old_w)

    @pl.when(g == GSTEPS - 1)
    def _():
        ow = jax.lax.dot_general(
            qout_ref[...], bits_ref[pl.ds(pn, NWORD), :],
            (((1,), (0,)), ((), ())),
            precision=lax.Precision.HIGHEST,
            preferred_element_type=jnp.float32)
        ov = ow * oinv_ref[...]
        ofv = jnp.floor(ov)
        out_ref[...] = out_ref[...] + (ofv - 2.0 * jnp.floor(ofv * 0.5))


_tc_sim = pl.pallas_call(
    _tc_body,
    grid=(CYCLES, GSTEPS),
    in_specs=[
        pl.BlockSpec((1, CB, AXONS, B), lambda t, g: (t, g, 0, 0)),
        pl.BlockSpec((1, CB, NEURONS, AXONS), lambda t, g: (g, 0, 0, 0)),
        pl.BlockSpec((1, CB * AXONS, NWORD), lambda t, g: (g, 0, 0)),
        pl.BlockSpec((1, CB * AXONS, 1), lambda t, g: (g, 0, 0)),
        pl.BlockSpec((N_OUT, NWORD), lambda t, g: (0, 0)),
        pl.BlockSpec((N_OUT, 1), lambda t, g: (0, 0)),
        pl.BlockSpec((4, NEURONS), lambda t, g: (0, 0)),
        pl.BlockSpec((1, CB, 1, 2), lambda t, g: (t * GSTEPS + g, 0, 0, 0),
                     memory_space=pltpu.SMEM),
    ],
    out_specs=pl.BlockSpec((N_OUT, B), lambda t, g: (0, 0)),
    out_shape=jax.ShapeDtypeStruct((N_OUT, B), jnp.float32),
    scratch_shapes=[
        pltpu.VMEM((2 * NWORD, B), jnp.float32),
        pltpu.VMEM((N_CORES * NEURONS, B), jnp.float32),
    ],
    compiler_params=pltpu.CompilerParams(
        dimension_semantics=("arbitrary", "arbitrary")),
)


def kernel(x, core_params, thresholds, axon_idx, out_idx, latencies):
    base = jax.random.key(42)

    sps = []
    for t in range(CYCLES):
        k1 = jax.random.fold_in(base, 2 * t)
        sps.append(((jax.random.uniform(k1, x.shape) < x)
                    .astype(jnp.float32)).T)
    spk = jnp.stack(sps)
    const = jnp.concatenate([jnp.zeros((1, B), jnp.float32),
                             jnp.ones((1, B), jnp.float32),
                             jnp.zeros((6, B), jnp.float32)], axis=0)

    idx = axon_idx.reshape(-1)
    is_buf = (idx >= D_IN) & (idx < D_IN + NBUF)
    idx_s = jnp.where(is_buf, D_IN,
                      jnp.where(idx >= D_IN + NBUF, idx - NBUF,
                                idx)).astype(jnp.int32)
    word = jnp.where(is_buf, (idx - D_IN) // 16, NWORD)
    qb = (word[:, None] == jnp.arange(NWORD)[None, :]).astype(jnp.float32)
    qb = qb.reshape(GSTEPS, CB * AXONS, NWORD)
    shift = jnp.where(is_buf, (idx - D_IN) % 16, 0).astype(jnp.float32)
    inv2r = (2.0 ** (-shift)).reshape(GSTEPS, CB * AXONS, 1)

    oword = ((out_idx - D_IN) // 16).astype(jnp.int32)
    qout = (oword[:, None] == jnp.arange(NWORD)[None, :]).astype(jnp.float32)
    oinv = (2.0 ** (-((out_idx - D_IN) % 16).astype(jnp.float32)))[:, None]

    n_ar = jnp.arange(NEURONS)
    pmat = jnp.where((n_ar[None, :] // 16) == jnp.arange(4)[:, None],
                     2.0 ** (n_ar[None, :] % 16).astype(jnp.float32), 0.0)

    active = (jnp.arange(CYCLES, dtype=jnp.int32)[:, None]
              >= latencies[None, :]).astype(jnp.float32)
    scal = jnp.stack(
        [active, jnp.broadcast_to(thresholds[None, :], (CYCLES, N_CORES))],
        axis=-1).reshape(CYCLES * GSTEPS, CB, 1, 2)

    in_static = _sc_static(spk, const, idx_s)

    out_T = _tc_sim(in_static.reshape(CYCLES, N_CORES, AXONS, B),
                    core_params.reshape(GSTEPS, CB, NEURONS, AXONS),
                    qb, inv2r, qout, oinv, pmat, scal)
    return out_T.T


# speedup vs baseline: 1.1912x; 1.1912x over previous
import functools

import jax
import jax.numpy as jnp
from jax import lax
from jax.experimental import pallas as pl
from jax.experimental.pallas import tpu as pltpu
from jax.experimental.pallas import tpu_sc as plsc

B = 128
D_IN = 4096
N_CORES = 128
AXONS = 64
NEURONS = 64
SIM_LEN = 8
MAX_LAT = 3
CYCLES = MAX_LAT + SIM_LEN
N_OUT = 1024
NBUF = N_CORES * NEURONS
AX_TOT = N_CORES * AXONS
BANK_ROWS = D_IN + 8
NWORD = NBUF // 8
CB = 8
GSTEPS = N_CORES // CB

_MESH = plsc.VectorSubcoreMesh(core_axis_name="c", subcore_axis_name="s",
                               num_cores=2, num_subcores=16)


@functools.partial(
    pl.kernel,
    out_type=jax.ShapeDtypeStruct((CYCLES, AX_TOT, B), jnp.float32),
    mesh=_MESH,
    scratch_types=[
        pltpu.VMEM((128,), jnp.int32),
        pltpu.VMEM((128,), jnp.int32),
        pltpu.VMEM((256, B), jnp.float32),
        pltpu.VMEM_SHARED((BANK_ROWS, B), jnp.float32),
    ],
    compiler_params=pltpu.CompilerParams(use_tc_tiling_on_sc=False),
)
def _sc_static(spk_hbm, const_hbm, idx_hbm, out_hbm,
               idx_v0, idx_v1, rows_v, bank):
    tid = lax.axis_index("s")
    wid = tid * 2 + lax.axis_index("c")
    base = wid * (AX_TOT // 32)
    pltpu.sync_copy(idx_hbm.at[pl.ds(base, 128)], idx_v0)
    pltpu.sync_copy(idx_hbm.at[pl.ds(base + 128, 128)], idx_v1)

    def cycle_body(t, carry):
        pltpu.sync_copy(spk_hbm.at[t, pl.ds(tid * 256, 256)],
                        bank.at[pl.ds(tid * 256, 256)])
        @pl.when(tid == 0)
        def _():
            pltpu.sync_copy(const_hbm, bank.at[pl.ds(D_IN, 8)])
        plsc.subcore_barrier()
        pltpu.sync_copy(bank.at[idx_v0], rows_v.at[pl.ds(0, 128)])
        pltpu.sync_copy(bank.at[idx_v1], rows_v.at[pl.ds(128, 128)])
        pltpu.sync_copy(rows_v, out_hbm.at[t, pl.ds(base, 256)])
        plsc.subcore_barrier()
        return carry

    lax.fori_loop(0, CYCLES, cycle_body, 0)


def _tc_body(ins_ref, w_ref, qb_ref, inv2r_ref, qout_ref, oinv_ref,
             pmat_ref, scal_ref, out_ref, bits_ref, memb_ref):
    t = pl.program_id(0)
    g = pl.program_id(1)

    @pl.when(jnp.logical_and(t == 0, g == 0))
    def _():
        out_ref[...] = jnp.zeros((N_OUT, B), jnp.float32)

    pr = (t % 2) * NWORD
    pn = ((t + 1) % 2) * NWORD
    bits_eff = jnp.where(t == 0, 0.0, bits_ref[pl.ds(pr, NWORD), :])
    qb = qb_ref[g]
    words = jax.lax.dot_general(
        qb, bits_eff.astype(jnp.bfloat16), (((1,), (0,)), ((), ())),
        preferred_element_type=jnp.float32)
    v = words * inv2r_ref[0]
    fv = jnp.floor(v)
    buf_bit = fv - 2.0 * jnp.floor(fv * 0.5)
    in3 = ins_ref[0].reshape(CB * AXONS, B) + buf_bit

    pm = pmat_ref[...]
    for ci in range(CB):
        a = scal_ref[0, ci, 0, 0]
        thr = scal_ref[0, ci, 0, 1]
        act = a > 0.5
        x = in3[ci * AXONS:(ci + 1) * AXONS, :]
        delta = jnp.dot(w_ref[0, ci], x, preferred_element_type=jnp.float32)
        mrow = pl.ds(ci * NEURONS + g * CB * NEURONS, NEURONS)
        mold = jnp.where(t == 0, 0.0, memb_ref[mrow, :])
        mn = mold + a * delta
        fb = mn > thr
        fired = jnp.where(fb, 1.0, 0.0)
        memb_ref[mrow, :] = jnp.where(jnp.logical_and(act, fb), 0.0, mn)
        packed = jnp.dot(pm, fired, preferred_element_type=jnp.float32)
        wrow = (g * CB + ci) * 8
        old_w = jnp.where(t == 0, 0.0, bits_ref[pl.ds(pr + wrow, 8), :])
        bits_ref[pl.ds(pn + wrow, 8), :] = jnp.where(act, packed, old_w)

    @pl.when(g == GSTEPS - 1)
    def _():
        ow = jax.lax.dot_general(
            qout_ref[...], bits_ref[pl.ds(pn, NWORD), :].astype(jnp.bfloat16),
            (((1,), (0,)), ((), ())),
            preferred_element_type=jnp.float32)
        ov = ow * oinv_ref[...]
        ofv = jnp.floor(ov)
        out_ref[...] = out_ref[...] + (ofv - 2.0 * jnp.floor(ofv * 0.5))


_tc_sim = pl.pallas_call(
    _tc_body,
    grid=(CYCLES, GSTEPS),
    in_specs=[
        pl.BlockSpec((1, CB, AXONS, B), lambda t, g: (t, g, 0, 0)),
        pl.BlockSpec((1, CB, NEURONS, AXONS), lambda t, g: (g, 0, 0, 0)),
        pl.BlockSpec((GSTEPS, CB * AXONS, NWORD), lambda t, g: (0, 0, 0)),
        pl.BlockSpec((1, CB * AXONS, 1), lambda t, g: (g, 0, 0)),
        pl.BlockSpec((N_OUT, NWORD), lambda t, g: (0, 0)),
        pl.BlockSpec((N_OUT, 1), lambda t, g: (0, 0)),
        pl.BlockSpec((8, NEURONS), lambda t, g: (0, 0)),
        pl.BlockSpec((1, CB, 1, 2), lambda t, g: (t * GSTEPS + g, 0, 0, 0),
                     memory_space=pltpu.SMEM),
    ],
    out_specs=pl.BlockSpec((N_OUT, B), lambda t, g: (0, 0)),
    out_shape=jax.ShapeDtypeStruct((N_OUT, B), jnp.float32),
    scratch_shapes=[
        pltpu.VMEM((2 * NWORD, B), jnp.float32),
        pltpu.VMEM((N_CORES * NEURONS, B), jnp.float32),
    ],
    compiler_params=pltpu.CompilerParams(
        dimension_semantics=("arbitrary", "arbitrary")),
)


def kernel(x, core_params, thresholds, axon_idx, out_idx, latencies):
    base = jax.random.key(42)

    sps = []
    for t in range(CYCLES):
        k1 = jax.random.fold_in(base, 2 * t)
        sps.append(((jax.random.uniform(k1, x.shape) < x)
                    .astype(jnp.float32)).T)
    spk = jnp.stack(sps)
    const = jnp.concatenate([jnp.zeros((1, B), jnp.float32),
                             jnp.ones((1, B), jnp.float32),
                             jnp.zeros((6, B), jnp.float32)], axis=0)

    idx = axon_idx.reshape(-1)
    is_buf = (idx >= D_IN) & (idx < D_IN + NBUF)
    idx_s = jnp.where(is_buf, D_IN,
                      jnp.where(idx >= D_IN + NBUF, idx - NBUF,
                                idx)).astype(jnp.int32)
    word = jnp.where(is_buf, (idx - D_IN) // 8, NWORD)
    qb = (word[:, None] == jnp.arange(NWORD)[None, :]).astype(jnp.bfloat16)
    qb = qb.reshape(GSTEPS, CB * AXONS, NWORD)
    shift = jnp.where(is_buf, (idx - D_IN) % 8, 0).astype(jnp.float32)
    inv2r = (2.0 ** (-shift)).reshape(GSTEPS, CB * AXONS, 1)

    oword = ((out_idx - D_IN) // 8).astype(jnp.int32)
    qout = (oword[:, None] == jnp.arange(NWORD)[None, :]).astype(jnp.bfloat16)
    oinv = (2.0 ** (-((out_idx - D_IN) % 8).astype(jnp.float32)))[:, None]

    n_ar = jnp.arange(NEURONS)
    pmat = jnp.where((n_ar[None, :] // 8) == jnp.arange(8)[:, None],
                     2.0 ** (n_ar[None, :] % 8).astype(jnp.float32), 0.0)

    active = (jnp.arange(CYCLES, dtype=jnp.int32)[:, None]
              >= latencies[None, :]).astype(jnp.float32)
    scal = jnp.stack(
        [active, jnp.broadcast_to(thresholds[None, :], (CYCLES, N_CORES))],
        axis=-1).reshape(CYCLES * GSTEPS, CB, 1, 2)

    in_static = _sc_static(spk, const, idx_s)

    out_T = _tc_sim(in_static.reshape(CYCLES, N_CORES, AXONS, B),
                    core_params.reshape(GSTEPS, CB, NEURONS, AXONS),
                    qb, inv2r, qout, oinv, pmat, scal)
    return out_T.T


# speedup vs baseline: 1.2567x; 1.0550x over previous
import functools

import jax
import jax.numpy as jnp
from jax import lax
from jax.experimental import pallas as pl
from jax.experimental.pallas import tpu as pltpu
from jax.experimental.pallas import tpu_sc as plsc

B = 128
D_IN = 4096
N_CORES = 128
AXONS = 64
NEURONS = 64
SIM_LEN = 8
MAX_LAT = 3
CYCLES = MAX_LAT + SIM_LEN
N_OUT = 1024
NBUF = N_CORES * NEURONS
AX_TOT = N_CORES * AXONS
BANK_ROWS = D_IN + 8
NWORD = NBUF // 8
CB = 16
GSTEPS = N_CORES // CB

_MESH = plsc.VectorSubcoreMesh(core_axis_name="c", subcore_axis_name="s",
                               num_cores=2, num_subcores=16)


@functools.partial(
    pl.kernel,
    out_type=jax.ShapeDtypeStruct((CYCLES, AX_TOT, B), jnp.float32),
    mesh=_MESH,
    scratch_types=[
        pltpu.VMEM((128,), jnp.int32),
        pltpu.VMEM((128,), jnp.int32),
        pltpu.VMEM((256, B), jnp.float32),
        pltpu.VMEM_SHARED((BANK_ROWS, B), jnp.float32),
    ],
    compiler_params=pltpu.CompilerParams(use_tc_tiling_on_sc=False),
)
def _sc_static(spk_hbm, const_hbm, idx_hbm, out_hbm,
               idx_v0, idx_v1, rows_v, bank):
    tid = lax.axis_index("s")
    wid = tid * 2 + lax.axis_index("c")
    base = wid * (AX_TOT // 32)
    pltpu.sync_copy(idx_hbm.at[pl.ds(base, 128)], idx_v0)
    pltpu.sync_copy(idx_hbm.at[pl.ds(base + 128, 128)], idx_v1)

    def cycle_body(t, carry):
        pltpu.sync_copy(spk_hbm.at[t, pl.ds(tid * 256, 256)],
                        bank.at[pl.ds(tid * 256, 256)])
        @pl.when(tid == 0)
        def _():
            pltpu.sync_copy(const_hbm, bank.at[pl.ds(D_IN, 8)])
        plsc.subcore_barrier()
        pltpu.sync_copy(bank.at[idx_v0], rows_v.at[pl.ds(0, 128)])
        pltpu.sync_copy(bank.at[idx_v1], rows_v.at[pl.ds(128, 128)])
        pltpu.sync_copy(rows_v, out_hbm.at[t, pl.ds(base, 256)])
        plsc.subcore_barrier()
        return carry

    lax.fori_loop(0, CYCLES, cycle_body, 0)


def _tc_body(ins_ref, w_ref, qb_ref, inv2r_ref, qout_ref, oinv_ref,
             pmat_ref, scal_ref, out_ref, bits_ref, memb_ref):
    t = pl.program_id(0)
    g = pl.program_id(1)

    @pl.when(jnp.logical_and(t == 0, g == 0))
    def _():
        out_ref[...] = jnp.zeros((N_OUT, B), jnp.float32)

    pr = (t % 2) * NWORD
    pn = ((t + 1) % 2) * NWORD
    bits_eff = jnp.where(t == 0, 0.0, bits_ref[pl.ds(pr, NWORD), :])
    qb = qb_ref[g]
    words = jax.lax.dot_general(
        qb, bits_eff.astype(jnp.bfloat16), (((1,), (0,)), ((), ())),
        preferred_element_type=jnp.float32)
    v = words * inv2r_ref[0]
    fv = jnp.floor(v)
    buf_bit = fv - 2.0 * jnp.floor(fv * 0.5)
    in3 = ins_ref[0].reshape(CB * AXONS, B) + buf_bit

    pm = pmat_ref[...]
    for ci in range(CB):
        a = scal_ref[0, ci, 0, 0]
        thr = scal_ref[0, ci, 0, 1]
        act = a > 0.5
        x = in3[ci * AXONS:(ci + 1) * AXONS, :]
        delta = jnp.dot(w_ref[0, ci], x, preferred_element_type=jnp.float32)
        mrow = pl.ds(ci * NEURONS + g * CB * NEURONS, NEURONS)
        mold = jnp.where(t == 0, 0.0, memb_ref[mrow, :])
        mn = mold + a * delta
        fb = mn > thr
        fired = jnp.where(fb, 1.0, 0.0)
        memb_ref[mrow, :] = jnp.where(jnp.logical_and(act, fb), 0.0, mn)
        packed = jnp.dot(pm, fired, preferred_element_type=jnp.float32)
        wrow = (g * CB + ci) * 8
        old_w = jnp.where(t == 0, 0.0, bits_ref[pl.ds(pr + wrow, 8), :])
        bits_ref[pl.ds(pn + wrow, 8), :] = jnp.where(act, packed, old_w)

    @pl.when(g == GSTEPS - 1)
    def _():
        ow = jax.lax.dot_general(
            qout_ref[...], bits_ref[pl.ds(pn, NWORD), :].astype(jnp.bfloat16),
            (((1,), (0,)), ((), ())),
            preferred_element_type=jnp.float32)
        ov = ow * oinv_ref[...]
        ofv = jnp.floor(ov)
        out_ref[...] = out_ref[...] + (ofv - 2.0 * jnp.floor(ofv * 0.5))


_tc_sim = pl.pallas_call(
    _tc_body,
    grid=(CYCLES, GSTEPS),
    in_specs=[
        pl.BlockSpec((1, CB, AXONS, B), lambda t, g: (t, g, 0, 0)),
        pl.BlockSpec((1, CB, NEURONS, AXONS), lambda t, g: (g, 0, 0, 0)),
        pl.BlockSpec((GSTEPS, CB * AXONS, NWORD), lambda t, g: (0, 0, 0)),
        pl.BlockSpec((1, CB * AXONS, 1), lambda t, g: (g, 0, 0)),
        pl.BlockSpec((N_OUT, NWORD), lambda t, g: (0, 0)),
        pl.BlockSpec((N_OUT, 1), lambda t, g: (0, 0)),
        pl.BlockSpec((8, NEURONS), lambda t, g: (0, 0)),
        pl.BlockSpec((1, CB, 1, 2), lambda t, g: (t * GSTEPS + g, 0, 0, 0),
                     memory_space=pltpu.SMEM),
    ],
    out_specs=pl.BlockSpec((N_OUT, B), lambda t, g: (0, 0)),
    out_shape=jax.ShapeDtypeStruct((N_OUT, B), jnp.float32),
    scratch_shapes=[
        pltpu.VMEM((2 * NWORD, B), jnp.float32),
        pltpu.VMEM((N_CORES * NEURONS, B), jnp.float32),
    ],
    compiler_params=pltpu.CompilerParams(
        dimension_semantics=("arbitrary", "arbitrary")),
)


def kernel(x, core_params, thresholds, axon_idx, out_idx, latencies):
    base = jax.random.key(42)

    sps = []
    for t in range(CYCLES):
        k1 = jax.random.fold_in(base, 2 * t)
        sps.append(((jax.random.uniform(k1, x.shape) < x)
                    .astype(jnp.float32)).T)
    spk = jnp.stack(sps)
    const = jnp.concatenate([jnp.zeros((1, B), jnp.float32),
                             jnp.ones((1, B), jnp.float32),
                             jnp.zeros((6, B), jnp.float32)], axis=0)

    idx = axon_idx.reshape(-1)
    is_buf = (idx >= D_IN) & (idx < D_IN + NBUF)
    idx_s = jnp.where(is_buf, D_IN,
                      jnp.where(idx >= D_IN + NBUF, idx - NBUF,
                                idx)).astype(jnp.int32)
    word = jnp.where(is_buf, (idx - D_IN) // 8, NWORD)
    qb = (word[:, None] == jnp.arange(NWORD)[None, :]).astype(jnp.bfloat16)
    qb = qb.reshape(GSTEPS, CB * AXONS, NWORD)
    shift = jnp.where(is_buf, (idx - D_IN) % 8, 0).astype(jnp.float32)
    inv2r = (2.0 ** (-shift)).reshape(GSTEPS, CB * AXONS, 1)

    oword = ((out_idx - D_IN) // 8).astype(jnp.int32)
    qout = (oword[:, None] == jnp.arange(NWORD)[None, :]).astype(jnp.bfloat16)
    oinv = (2.0 ** (-((out_idx - D_IN) % 8).astype(jnp.float32)))[:, None]

    n_ar = jnp.arange(NEURONS)
    pmat = jnp.where((n_ar[None, :] // 8) == jnp.arange(8)[:, None],
                     2.0 ** (n_ar[None, :] % 8).astype(jnp.float32), 0.0)

    active = (jnp.arange(CYCLES, dtype=jnp.int32)[:, None]
              >= latencies[None, :]).astype(jnp.float32)
    scal = jnp.stack(
        [active, jnp.broadcast_to(thresholds[None, :], (CYCLES, N_CORES))],
        axis=-1).reshape(CYCLES * GSTEPS, CB, 1, 2)

    in_static = _sc_static(spk, const, idx_s)

    out_T = _tc_sim(in_static.reshape(CYCLES, N_CORES, AXONS, B),
                    core_params.reshape(GSTEPS, CB, NEURONS, AXONS),
                    qb, inv2r, qout, oinv, pmat, scal)
    return out_T.T


# speedup vs baseline: 1.2900x; 1.0266x over previous
import functools

import jax
import jax.numpy as jnp
from jax import lax
from jax.experimental import pallas as pl
from jax.experimental.pallas import tpu as pltpu
from jax.experimental.pallas import tpu_sc as plsc

B = 128
D_IN = 4096
N_CORES = 128
AXONS = 64
NEURONS = 64
SIM_LEN = 8
MAX_LAT = 3
CYCLES = MAX_LAT + SIM_LEN
N_OUT = 1024
NBUF = N_CORES * NEURONS
AX_TOT = N_CORES * AXONS
BANK_ROWS = D_IN + 8
NWORD = NBUF // 8
CB = 32
GSTEPS = N_CORES // CB

_MESH = plsc.VectorSubcoreMesh(core_axis_name="c", subcore_axis_name="s",
                               num_cores=2, num_subcores=16)


@functools.partial(
    pl.kernel,
    out_type=jax.ShapeDtypeStruct((CYCLES, AX_TOT, B), jnp.float32),
    mesh=_MESH,
    scratch_types=[
        pltpu.VMEM((128,), jnp.int32),
        pltpu.VMEM((128,), jnp.int32),
        pltpu.VMEM((256, B), jnp.float32),
        pltpu.VMEM_SHARED((BANK_ROWS, B), jnp.float32),
    ],
    compiler_params=pltpu.CompilerParams(use_tc_tiling_on_sc=False),
)
def _sc_static(spk_hbm, const_hbm, idx_hbm, out_hbm,
               idx_v0, idx_v1, rows_v, bank):
    tid = lax.axis_index("s")
    wid = tid * 2 + lax.axis_index("c")
    base = wid * (AX_TOT // 32)
    pltpu.sync_copy(idx_hbm.at[pl.ds(base, 128)], idx_v0)
    pltpu.sync_copy(idx_hbm.at[pl.ds(base + 128, 128)], idx_v1)

    def cycle_body(t, carry):
        pltpu.sync_copy(spk_hbm.at[t, pl.ds(tid * 256, 256)],
                        bank.at[pl.ds(tid * 256, 256)])
        @pl.when(tid == 0)
        def _():
            pltpu.sync_copy(const_hbm, bank.at[pl.ds(D_IN, 8)])
        plsc.subcore_barrier()
        pltpu.sync_copy(bank.at[idx_v0], rows_v.at[pl.ds(0, 128)])
        pltpu.sync_copy(bank.at[idx_v1], rows_v.at[pl.ds(128, 128)])
        pltpu.sync_copy(rows_v, out_hbm.at[t, pl.ds(base, 256)])
        plsc.subcore_barrier()
        return carry

    lax.fori_loop(0, CYCLES, cycle_body, 0)


def _tc_body(ins_ref, w_ref, qb_ref, inv2r_ref, qout_ref, oinv_ref,
             pmat_ref, scal_ref, out_ref, bits_ref, memb_ref):
    t = pl.program_id(0)
    g = pl.program_id(1)

    @pl.when(jnp.logical_and(t == 0, g == 0))
    def _():
        out_ref[...] = jnp.zeros((N_OUT, B), jnp.float32)

    pr = (t % 2) * NWORD
    pn = ((t + 1) % 2) * NWORD
    bits_eff = jnp.where(t == 0, 0.0, bits_ref[pl.ds(pr, NWORD), :])
    qb = qb_ref[g]
    words = jax.lax.dot_general(
        qb, bits_eff.astype(jnp.bfloat16), (((1,), (0,)), ((), ())),
        preferred_element_type=jnp.float32)
    v = words * inv2r_ref[0]
    fv = jnp.floor(v)
    buf_bit = fv - 2.0 * jnp.floor(fv * 0.5)
    in3 = ins_ref[0].reshape(CB * AXONS, B) + buf_bit

    pm = pmat_ref[...]
    for ci in range(CB):
        a = scal_ref[0, ci, 0, 0]
        thr = scal_ref[0, ci, 0, 1]
        act = a > 0.5
        x = in3[ci * AXONS:(ci + 1) * AXONS, :]
        delta = jnp.dot(w_ref[0, ci], x, preferred_element_type=jnp.float32)
        mrow = pl.ds(ci * NEURONS + g * CB * NEURONS, NEURONS)
        mold = jnp.where(t == 0, 0.0, memb_ref[mrow, :])
        mn = mold + a * delta
        fb = mn > thr
        fired = jnp.where(fb, 1.0, 0.0)
        memb_ref[mrow, :] = jnp.where(jnp.logical_and(act, fb), 0.0, mn)
        packed = jnp.dot(pm, fired, preferred_element_type=jnp.float32)
        wrow = (g * CB + ci) * 8
        old_w = jnp.where(t == 0, 0.0, bits_ref[pl.ds(pr + wrow, 8), :])
        bits_ref[pl.ds(pn + wrow, 8), :] = jnp.where(act, packed, old_w)

    @pl.when(g == GSTEPS - 1)
    def _():
        ow = jax.lax.dot_general(
            qout_ref[...], bits_ref[pl.ds(pn, NWORD), :].astype(jnp.bfloat16),
            (((1,), (0,)), ((), ())),
            preferred_element_type=jnp.float32)
        ov = ow * oinv_ref[...]
        ofv = jnp.floor(ov)
        out_ref[...] = out_ref[...] + (ofv - 2.0 * jnp.floor(ofv * 0.5))


_tc_sim = pl.pallas_call(
    _tc_body,
    grid=(CYCLES, GSTEPS),
    in_specs=[
        pl.BlockSpec((1, CB, AXONS, B), lambda t, g: (t, g, 0, 0)),
        pl.BlockSpec((1, CB, NEURONS, AXONS), lambda t, g: (g, 0, 0, 0)),
        pl.BlockSpec((GSTEPS, CB * AXONS, NWORD), lambda t, g: (0, 0, 0)),
        pl.BlockSpec((1, CB * AXONS, 1), lambda t, g: (g, 0, 0)),
        pl.BlockSpec((N_OUT, NWORD), lambda t, g: (0, 0)),
        pl.BlockSpec((N_OUT, 1), lambda t, g: (0, 0)),
        pl.BlockSpec((8, NEURONS), lambda t, g: (0, 0)),
        pl.BlockSpec((1, CB, 1, 2), lambda t, g: (t * GSTEPS + g, 0, 0, 0),
                     memory_space=pltpu.SMEM),
    ],
    out_specs=pl.BlockSpec((N_OUT, B), lambda t, g: (0, 0)),
    out_shape=jax.ShapeDtypeStruct((N_OUT, B), jnp.float32),
    scratch_shapes=[
        pltpu.VMEM((2 * NWORD, B), jnp.float32),
        pltpu.VMEM((N_CORES * NEURONS, B), jnp.float32),
    ],
    compiler_params=pltpu.CompilerParams(
        dimension_semantics=("arbitrary", "arbitrary")),
)


def kernel(x, core_params, thresholds, axon_idx, out_idx, latencies):
    base = jax.random.key(42)

    sps = []
    for t in range(CYCLES):
        k1 = jax.random.fold_in(base, 2 * t)
        sps.append(((jax.random.uniform(k1, x.shape) < x)
                    .astype(jnp.float32)).T)
    spk = jnp.stack(sps)
    const = jnp.concatenate([jnp.zeros((1, B), jnp.float32),
                             jnp.ones((1, B), jnp.float32),
                             jnp.zeros((6, B), jnp.float32)], axis=0)

    idx = axon_idx.reshape(-1)
    is_buf = (idx >= D_IN) & (idx < D_IN + NBUF)
    idx_s = jnp.where(is_buf, D_IN,
                      jnp.where(idx >= D_IN + NBUF, idx - NBUF,
                                idx)).astype(jnp.int32)
    word = jnp.where(is_buf, (idx - D_IN) // 8, NWORD)
    qb = (word[:, None] == jnp.arange(NWORD)[None, :]).astype(jnp.bfloat16)
    qb = qb.reshape(GSTEPS, CB * AXONS, NWORD)
    shift = jnp.where(is_buf, (idx - D_IN) % 8, 0).astype(jnp.float32)
    inv2r = (2.0 ** (-shift)).reshape(GSTEPS, CB * AXONS, 1)

    oword = ((out_idx - D_IN) // 8).astype(jnp.int32)
    qout = (oword[:, None] == jnp.arange(NWORD)[None, :]).astype(jnp.bfloat16)
    oinv = (2.0 ** (-((out_idx - D_IN) % 8).astype(jnp.float32)))[:, None]

    n_ar = jnp.arange(NEURONS)
    pmat = jnp.where((n_ar[None, :] // 8) == jnp.arange(8)[:, None],
                     2.0 ** (n_ar[None, :] % 8).astype(jnp.float32), 0.0)

    active = (jnp.arange(CYCLES, dtype=jnp.int32)[:, None]
              >= latencies[None, :]).astype(jnp.float32)
    scal = jnp.stack(
        [active, jnp.broadcast_to(thresholds[None, :], (CYCLES, N_CORES))],
        axis=-1).reshape(CYCLES * GSTEPS, CB, 1, 2)

    in_static = _sc_static(spk, const, idx_s)

    out_T = _tc_sim(in_static.reshape(CYCLES, N_CORES, AXONS, B),
                    core_params.reshape(GSTEPS, CB, NEURONS, AXONS),
                    qb, inv2r, qout, oinv, pmat, scal)
    return out_T.T


# speedup vs baseline: 1.3074x; 1.0134x over previous
import functools

import jax
import jax.numpy as jnp
from jax import lax
from jax.experimental import pallas as pl
from jax.experimental.pallas import tpu as pltpu
from jax.experimental.pallas import tpu_sc as plsc

B = 128
D_IN = 4096
N_CORES = 128
AXONS = 64
NEURONS = 64
SIM_LEN = 8
MAX_LAT = 3
CYCLES = MAX_LAT + SIM_LEN
N_OUT = 1024
NBUF = N_CORES * NEURONS
AX_TOT = N_CORES * AXONS
BANK_ROWS = D_IN + 8
NWORD = NBUF // 8
CB = 64
GSTEPS = N_CORES // CB

_MESH = plsc.VectorSubcoreMesh(core_axis_name="c", subcore_axis_name="s",
                               num_cores=2, num_subcores=16)


@functools.partial(
    pl.kernel,
    out_type=jax.ShapeDtypeStruct((CYCLES, AX_TOT, B), jnp.float32),
    mesh=_MESH,
    scratch_types=[
        pltpu.VMEM((128,), jnp.int32),
        pltpu.VMEM((128,), jnp.int32),
        pltpu.VMEM((256, B), jnp.float32),
        pltpu.VMEM_SHARED((BANK_ROWS, B), jnp.float32),
    ],
    compiler_params=pltpu.CompilerParams(use_tc_tiling_on_sc=False),
)
def _sc_static(spk_hbm, const_hbm, idx_hbm, out_hbm,
               idx_v0, idx_v1, rows_v, bank):
    tid = lax.axis_index("s")
    wid = tid * 2 + lax.axis_index("c")
    base = wid * (AX_TOT // 32)
    pltpu.sync_copy(idx_hbm.at[pl.ds(base, 128)], idx_v0)
    pltpu.sync_copy(idx_hbm.at[pl.ds(base + 128, 128)], idx_v1)

    def cycle_body(t, carry):
        pltpu.sync_copy(spk_hbm.at[t, pl.ds(tid * 256, 256)],
                        bank.at[pl.ds(tid * 256, 256)])
        @pl.when(tid == 0)
        def _():
            pltpu.sync_copy(const_hbm, bank.at[pl.ds(D_IN, 8)])
        plsc.subcore_barrier()
        pltpu.sync_copy(bank.at[idx_v0], rows_v.at[pl.ds(0, 128)])
        pltpu.sync_copy(bank.at[idx_v1], rows_v.at[pl.ds(128, 128)])
        pltpu.sync_copy(rows_v, out_hbm.at[t, pl.ds(base, 256)])
        plsc.subcore_barrier()
        return carry

    lax.fori_loop(0, CYCLES, cycle_body, 0)


def _tc_body(ins_ref, w_ref, qb_ref, inv2r_ref, qout_ref, oinv_ref,
             pmat_ref, scal_ref, out_ref, bits_ref, memb_ref):
    t = pl.program_id(0)
    g = pl.program_id(1)

    @pl.when(jnp.logical_and(t == 0, g == 0))
    def _():
        out_ref[...] = jnp.zeros((N_OUT, B), jnp.float32)

    pr = (t % 2) * NWORD
    pn = ((t + 1) % 2) * NWORD
    bits_eff = jnp.where(t == 0, 0.0, bits_ref[pl.ds(pr, NWORD), :])
    qb = qb_ref[g]
    words = jax.lax.dot_general(
        qb, bits_eff.astype(jnp.bfloat16), (((1,), (0,)), ((), ())),
        preferred_element_type=jnp.float32)
    v = words * inv2r_ref[0]
    fv = jnp.floor(v)
    buf_bit = fv - 2.0 * jnp.floor(fv * 0.5)
    in3 = ins_ref[0].reshape(CB * AXONS, B) + buf_bit

    pm = pmat_ref[...]
    for ci in range(CB):
        a = scal_ref[0, ci, 0, 0]
        thr = scal_ref[0, ci, 0, 1]
        act = a > 0.5
        x = in3[ci * AXONS:(ci + 1) * AXONS, :]
        delta = jnp.dot(w_ref[0, ci], x, preferred_element_type=jnp.float32)
        mrow = pl.ds(ci * NEURONS + g * CB * NEURONS, NEURONS)
        mold = jnp.where(t == 0, 0.0, memb_ref[mrow, :])
        mn = mold + a * delta
        fb = mn > thr
        fired = jnp.where(fb, 1.0, 0.0)
        memb_ref[mrow, :] = jnp.where(jnp.logical_and(act, fb), 0.0, mn)
        packed = jnp.dot(pm, fired, preferred_element_type=jnp.float32)
        wrow = (g * CB + ci) * 8
        old_w = jnp.where(t == 0, 0.0, bits_ref[pl.ds(pr + wrow, 8), :])
        bits_ref[pl.ds(pn + wrow, 8), :] = jnp.where(act, packed, old_w)

    @pl.when(g == GSTEPS - 1)
    def _():
        ow = jax.lax.dot_general(
            qout_ref[...], bits_ref[pl.ds(pn, NWORD), :].astype(jnp.bfloat16),
            (((1,), (0,)), ((), ())),
            preferred_element_type=jnp.float32)
        ov = ow * oinv_ref[...]
        ofv = jnp.floor(ov)
        out_ref[...] = out_ref[...] + (ofv - 2.0 * jnp.floor(ofv * 0.5))


_tc_sim = pl.pallas_call(
    _tc_body,
    grid=(CYCLES, GSTEPS),
    in_specs=[
        pl.BlockSpec((1, CB, AXONS, B), lambda t, g: (t, g, 0, 0)),
        pl.BlockSpec((1, CB, NEURONS, AXONS), lambda t, g: (g, 0, 0, 0)),
        pl.BlockSpec((GSTEPS, CB * AXONS, NWORD), lambda t, g: (0, 0, 0)),
        pl.BlockSpec((1, CB * AXONS, 1), lambda t, g: (g, 0, 0)),
        pl.BlockSpec((N_OUT, NWORD), lambda t, g: (0, 0)),
        pl.BlockSpec((N_OUT, 1), lambda t, g: (0, 0)),
        pl.BlockSpec((8, NEURONS), lambda t, g: (0, 0)),
        pl.BlockSpec((1, CB, 1, 2), lambda t, g: (t * GSTEPS + g, 0, 0, 0),
                     memory_space=pltpu.SMEM),
    ],
    out_specs=pl.BlockSpec((N_OUT, B), lambda t, g: (0, 0)),
    out_shape=jax.ShapeDtypeStruct((N_OUT, B), jnp.float32),
    scratch_shapes=[
        pltpu.VMEM((2 * NWORD, B), jnp.float32),
        pltpu.VMEM((N_CORES * NEURONS, B), jnp.float32),
    ],
    compiler_params=pltpu.CompilerParams(
        dimension_semantics=("arbitrary", "arbitrary")),
)


def kernel(x, core_params, thresholds, axon_idx, out_idx, latencies):
    base = jax.random.key(42)

    sps = []
    for t in range(CYCLES):
        k1 = jax.random.fold_in(base, 2 * t)
        sps.append(((jax.random.uniform(k1, x.shape) < x)
                    .astype(jnp.float32)).T)
    spk = jnp.stack(sps)
    const = jnp.concatenate([jnp.zeros((1, B), jnp.float32),
                             jnp.ones((1, B), jnp.float32),
                             jnp.zeros((6, B), jnp.float32)], axis=0)

    idx = axon_idx.reshape(-1)
    is_buf = (idx >= D_IN) & (idx < D_IN + NBUF)
    idx_s = jnp.where(is_buf, D_IN,
                      jnp.where(idx >= D_IN + NBUF, idx - NBUF,
                                idx)).astype(jnp.int32)
    word = jnp.where(is_buf, (idx - D_IN) // 8, NWORD)
    qb = (word[:, None] == jnp.arange(NWORD)[None, :]).astype(jnp.bfloat16)
    qb = qb.reshape(GSTEPS, CB * AXONS, NWORD)
    shift = jnp.where(is_buf, (idx - D_IN) % 8, 0).astype(jnp.float32)
    inv2r = (2.0 ** (-shift)).reshape(GSTEPS, CB * AXONS, 1)

    oword = ((out_idx - D_IN) // 8).astype(jnp.int32)
    qout = (oword[:, None] == jnp.arange(NWORD)[None, :]).astype(jnp.bfloat16)
    oinv = (2.0 ** (-((out_idx - D_IN) % 8).astype(jnp.float32)))[:, None]

    n_ar = jnp.arange(NEURONS)
    pmat = jnp.where((n_ar[None, :] // 8) == jnp.arange(8)[:, None],
                     2.0 ** (n_ar[None, :] % 8).astype(jnp.float32), 0.0)

    active = (jnp.arange(CYCLES, dtype=jnp.int32)[:, None]
              >= latencies[None, :]).astype(jnp.float32)
    scal = jnp.stack(
        [active, jnp.broadcast_to(thresholds[None, :], (CYCLES, N_CORES))],
        axis=-1).reshape(CYCLES * GSTEPS, CB, 1, 2)

    in_static = _sc_static(spk, const, idx_s)

    out_T = _tc_sim(in_static.reshape(CYCLES, N_CORES, AXONS, B),
                    core_params.reshape(GSTEPS, CB, NEURONS, AXONS),
                    qb, inv2r, qout, oinv, pmat, scal)
    return out_T.T
